# baseline (speedup 1.0000x reference)
import functools

import jax
import jax.numpy as jnp
from jax import lax
from jax.experimental import pallas as pl
from jax.experimental.pallas import tpu as pltpu
from jax.experimental.pallas import tpu_sc as plsc

_MAX_REL = 32
_NIDX = 2 * _MAX_REL + 1
_LANES = 16


def kernel(x, rel_table):
    batch, seq_len, hidden = x.shape
    n_workers = 32
    n_cores = 2
    rows_per_w = seq_len // n_workers
    inv = 1.0 / seq_len
    nchunk = hidden // _LANES
    mesh = plsc.VectorSubcoreMesh(core_axis_name="c", subcore_axis_name="s")

    @functools.partial(
        pl.kernel,
        mesh=mesh,
        out_type=jax.ShapeDtypeStruct(x.shape, x.dtype),
        scratch_types=[
            pltpu.VMEM((_NIDX, hidden), jnp.float32),
            pltpu.VMEM((rows_per_w, hidden), jnp.float32),
            pltpu.VMEM((rows_per_w, hidden), jnp.float32),
            pltpu.VMEM((rows_per_w, hidden), jnp.float32),
            pltpu.VMEM((rows_per_w, hidden), jnp.float32),
            pltpu.SemaphoreType.DMA,
            pltpu.SemaphoreType.DMA,
            pltpu.SemaphoreType.DMA,
            pltpu.SemaphoreType.DMA,
            pltpu.SemaphoreType.DMA,
            pltpu.SemaphoreType.DMA,
        ],
    )
    def allsc(x_hbm, tab_hbm, out_hbm, tab_v, blk_v, xb0, xb1, xb2,
              si0, si1, si2, so0, so1, so2):
        wid = lax.axis_index("s") * n_cores + lax.axis_index("c")
        base = wid * rows_per_w
        xbufs = (xb0, xb1, xb2)
        sin = (si0, si1, si2)
        sout = (so0, so1, so2)

        in_h = [None, None, None]
        in_h[0] = pltpu.async_copy(
            x_hbm.at[0, pl.ds(base, rows_per_w), :], xb0, si0)
        in_h[1] = pltpu.async_copy(
            x_hbm.at[1, pl.ds(base, rows_per_w), :], xb1, si1)

        pltpu.sync_copy(tab_hbm, tab_v)

        a0 = jnp.minimum(base, _MAX_REL - 1) + _MAX_REL + 1
        chi0 = (jnp.maximum(0, base - (_MAX_REL - 1))
                .astype(jnp.float32) * inv)
        clo0 = (jnp.maximum(0, (seq_len - _MAX_REL) - base)
                .astype(jnp.float32) * inv)

        @plsc.parallel_loop(0, nchunk)
        def _chunk(c):
            sl = pl.ds(c * _LANES, _LANES)
            zero = jnp.zeros((_LANES,), jnp.float32)
            acc = zero
            for k in range(1, _NIDX - 1):
                acc = acc + jnp.where(k < a0, tab_v[k, sl], zero)
            v = (acc * inv + chi0 * tab_v[_NIDX - 1, sl]
                 + clo0 * tab_v[0, sl])
            blk_v[0, sl] = v
            for jj in range(1, rows_per_w):
                j = base + jj
                hi_idx = jnp.minimum(j + _MAX_REL, _NIDX - 1)
                lo_idx = jnp.maximum(0, j - (seq_len - _MAX_REL))
                v = v + inv * (tab_v[hi_idx, sl] - tab_v[lo_idx, sl])
                blk_v[jj, sl] = v

        out_h = [None, None, None]
        for b in range(batch):
            cur = b % 3
            if b + 2 < batch:
                tgt = (b + 2) % 3
                if out_h[tgt] is not None:
                    out_h[tgt].wait()
                in_h[tgt] = pltpu.async_copy(
                    x_hbm.at[b + 2, pl.ds(base, rows_per_w), :],
                    xbufs[tgt], sin[tgt])
            in_h[cur].wait()
            buf = xbufs[cur]

            @plsc.parallel_loop(0, rows_per_w, unroll=2)
            def _add(r, buf=buf):
                for c in range(nchunk):
                    sl = pl.ds(c * _LANES, _LANES)
                    buf[r, sl] = buf[r, sl] + blk_v[r, sl]

            out_h[cur] = pltpu.async_copy(
                buf, out_hbm.at[b, pl.ds(base, rows_per_w), :], sout[cur])
        for h in out_h:
            h.wait()

    return allsc(x, rel_table)


# speedup vs baseline: 1.1840x; 1.1840x over previous
import functools

import jax
import jax.numpy as jnp
from jax import lax
from jax.experimental import pallas as pl
from jax.experimental.pallas import tpu as pltpu
from jax.experimental.pallas import tpu_sc as plsc

_MAX_REL = 32
_NIDX = 2 * _MAX_REL + 1
_LANES = 16
_NBUF = 4


def kernel(x, rel_table):
    batch, seq_len, hidden = x.shape
    n_workers = 32
    n_cores = 2
    rows_per_w = seq_len // n_workers
    inv = 1.0 / seq_len
    nchunk = hidden // _LANES
    mesh = plsc.VectorSubcoreMesh(core_axis_name="c", subcore_axis_name="s")

    @functools.partial(
        pl.kernel,
        mesh=mesh,
        out_type=jax.ShapeDtypeStruct(x.shape, x.dtype),
        scratch_types=(
            [pltpu.VMEM((_NIDX, hidden), jnp.float32)]
            + [pltpu.VMEM((rows_per_w, hidden), jnp.float32)]
            + [pltpu.VMEM((rows_per_w, hidden), jnp.float32)
               for _ in range(_NBUF)]
            + [pltpu.SemaphoreType.DMA for _ in range(2 * _NBUF)]
        ),
    )
    def allsc(x_hbm, tab_hbm, out_hbm, tab_v, blk_v, *bufs_sems):
        xbufs = bufs_sems[:_NBUF]
        sin = bufs_sems[_NBUF:2 * _NBUF]
        sout = bufs_sems[2 * _NBUF:]
        wid = lax.axis_index("s") * n_cores + lax.axis_index("c")
        base = wid * rows_per_w

        in_h = [None] * _NBUF
        out_h = [None] * _NBUF
        for i in range(_NBUF - 1):
            in_h[i] = pltpu.async_copy(
                x_hbm.at[i, pl.ds(base, rows_per_w), :], xbufs[i], sin[i])

        pltpu.sync_copy(tab_hbm, tab_v)

        a0 = jnp.minimum(base, _MAX_REL - 1) + _MAX_REL + 1
        chi0 = (jnp.maximum(0, base - (_MAX_REL - 1))
                .astype(jnp.float32) * inv)
        clo0 = (jnp.maximum(0, (seq_len - _MAX_REL) - base)
                .astype(jnp.float32) * inv)

        @plsc.parallel_loop(0, nchunk)
        def _chunk(c):
            sl = pl.ds(c * _LANES, _LANES)
            zero = jnp.zeros((_LANES,), jnp.float32)
            acc = zero
            for k in range(1, _NIDX - 1):
                acc = acc + jnp.where(k < a0, tab_v[k, sl], zero)
            v = (acc * inv + chi0 * tab_v[_NIDX - 1, sl]
                 + clo0 * tab_v[0, sl])
            blk_v[0, sl] = v
            for jj in range(1, rows_per_w):
                j = base + jj
                hi_idx = jnp.minimum(j + _MAX_REL, _NIDX - 1)
                lo_idx = jnp.maximum(0, j - (seq_len - _MAX_REL))
                v = v + inv * (tab_v[hi_idx, sl] - tab_v[lo_idx, sl])
                blk_v[jj, sl] = v

        for b in range(batch):
            cur = b % _NBUF
            if b + _NBUF - 1 < batch:
                tgt = (b + _NBUF - 1) % _NBUF
                if out_h[tgt] is not None:
                    out_h[tgt].wait()
                in_h[tgt] = pltpu.async_copy(
                    x_hbm.at[b + _NBUF - 1, pl.ds(base, rows_per_w), :],
                    xbufs[tgt], sin[tgt])
            in_h[cur].wait()
            buf = xbufs[cur]

            @plsc.parallel_loop(0, rows_per_w)
            def _add(r, buf=buf):
                for c in range(nchunk):
                    sl = pl.ds(c * _LANES, _LANES)
                    buf[r, sl] = buf[r, sl] + blk_v[r, sl]

            out_h[cur] = pltpu.async_copy(
                buf, out_hbm.at[b, pl.ds(base, rows_per_w), :], sout[cur])
        for h in out_h:
            if h is not None:
                h.wait()

    return allsc(x, rel_table)


# speedup vs baseline: 1.2651x; 1.0685x over previous
import functools

import jax
import jax.numpy as jnp
from jax import lax
from jax.experimental import pallas as pl
from jax.experimental.pallas import tpu as pltpu
from jax.experimental.pallas import tpu_sc as plsc

_MAX_REL = 32
_NIDX = 2 * _MAX_REL + 1
_LANES = 16
_NBUF = 4


def kernel(x, rel_table):
    batch, seq_len, hidden = x.shape
    n_workers = 32
    n_cores = 2
    rows_per_w = seq_len // n_workers
    inv = 1.0 / seq_len
    nchunk = hidden // _LANES
    mesh = plsc.VectorSubcoreMesh(core_axis_name="c", subcore_axis_name="s")

    @functools.partial(
        pl.kernel,
        mesh=mesh,
        out_type=jax.ShapeDtypeStruct(x.shape, x.dtype),
        scratch_types=(
            [pltpu.VMEM((_NIDX, hidden), jnp.float32)]
            + [pltpu.VMEM((rows_per_w, hidden), jnp.float32)]
            + [pltpu.VMEM((rows_per_w, hidden), jnp.float32)
               for _ in range(_NBUF)]
            + [pltpu.SemaphoreType.DMA for _ in range(2 * _NBUF)]
        ),
    )
    def allsc(x_hbm, tab_hbm, out_hbm, tab_v, blk_v, *bufs_sems):
        xbufs = bufs_sems[:_NBUF]
        sin = bufs_sems[_NBUF:2 * _NBUF]
        sout = bufs_sems[2 * _NBUF:]
        wid = lax.axis_index("s") * n_cores + lax.axis_index("c")
        base = wid * rows_per_w

        in_h = [None] * _NBUF
        out_h = [None] * _NBUF
        for i in range(2):
            in_h[i] = pltpu.async_copy(
                x_hbm.at[i, pl.ds(base, rows_per_w), :], xbufs[i], sin[i])

        pltpu.sync_copy(tab_hbm, tab_v)

        a0 = jnp.minimum(base, _MAX_REL - 1) + _MAX_REL + 1
        chi0 = (jnp.maximum(0, base - (_MAX_REL - 1))
                .astype(jnp.float32) * inv)
        clo0 = (jnp.maximum(0, (seq_len - _MAX_REL) - base)
                .astype(jnp.float32) * inv)

        @plsc.parallel_loop(0, nchunk)
        def _chunk(c):
            sl = pl.ds(c * _LANES, _LANES)
            zero = jnp.zeros((_LANES,), jnp.float32)
            acc = zero
            for k in range(1, _NIDX - 1):
                acc = acc + jnp.where(k < a0, tab_v[k, sl], zero)
            v = (acc * inv + chi0 * tab_v[_NIDX - 1, sl]
                 + clo0 * tab_v[0, sl])
            blk_v[0, sl] = v
            for jj in range(1, rows_per_w):
                j = base + jj
                hi_idx = jnp.minimum(j + _MAX_REL, _NIDX - 1)
                lo_idx = jnp.maximum(0, j - (seq_len - _MAX_REL))
                v = v + inv * (tab_v[hi_idx, sl] - tab_v[lo_idx, sl])
                blk_v[jj, sl] = v

        for p in range(batch // 2):
            s = (p % 2) * 2
            nb = 2 * (p + 1)
            if nb < batch:
                os = ((p + 1) % 2) * 2
                if out_h[os] is not None:
                    out_h[os].wait()
                    out_h[os + 1].wait()
                in_h[os] = pltpu.async_copy(
                    x_hbm.at[nb, pl.ds(base, rows_per_w), :],
                    xbufs[os], sin[os])
                in_h[os + 1] = pltpu.async_copy(
                    x_hbm.at[nb + 1, pl.ds(base, rows_per_w), :],
                    xbufs[os + 1], sin[os + 1])
            in_h[s].wait()
            in_h[s + 1].wait()
            b_a, b_b = xbufs[s], xbufs[s + 1]

            @plsc.parallel_loop(0, rows_per_w)
            def _add(r, b_a=b_a, b_b=b_b):
                for c in range(nchunk):
                    sl = pl.ds(c * _LANES, _LANES)
                    bv = blk_v[r, sl]
                    b_a[r, sl] = b_a[r, sl] + bv
                    b_b[r, sl] = b_b[r, sl] + bv

            out_h[s] = pltpu.async_copy(
                b_a, out_hbm.at[2 * p, pl.ds(base, rows_per_w), :], sout[s])
            out_h[s + 1] = pltpu.async_copy(
                b_b, out_hbm.at[2 * p + 1, pl.ds(base, rows_per_w), :],
                sout[s + 1])
        for h in out_h:
            if h is not None:
                h.wait()

    return allsc(x, rel_table)
